# baseline (speedup 1.0000x reference)
import functools

import jax
import jax.numpy as jnp
from jax import lax
from jax.experimental import pallas as pl
from jax.experimental.pallas import tpu as pltpu
from jax.experimental.pallas import tpu_sc as plsc

N = 10000
E = 320000
D = 128
G = 64

NUM_WORKERS = 32
BE = 64
EPT = E // NUM_WORKERS
NB = 160
EPT_PAD = NB * BE
NPAD = 10112
RPT = NPAD // 16

RB = 1000
GRID = N // RB


def _sc_segment_sum(h, srcp, dstp):
    mesh = plsc.VectorSubcoreMesh(core_axis_name="c", subcore_axis_name="s")

    @functools.partial(
        pl.kernel,
        out_type=jax.ShapeDtypeStruct((2, NPAD, 128), jnp.float32),
        mesh=mesh,
        scratch_types=[
            pltpu.VMEM_SHARED((NPAD, 128), jnp.float32),
            pltpu.VMEM((NB, BE), jnp.int32),
            pltpu.VMEM((BE,), jnp.int32),
            pltpu.VMEM((BE,), jnp.int32),
            pltpu.VMEM((BE, 128), jnp.float32),
            pltpu.VMEM((BE, 128), jnp.float32),
            pltpu.SemaphoreType.DMA,
            pltpu.SemaphoreType.DMA,
            pltpu.SemaphoreType.DMA,
            pltpu.SemaphoreType.DMA,
            pltpu.SemaphoreType.DMA,
            pltpu.SemaphoreType.DMA,
        ],
    )
    def segsum(h_hbm, src_hbm, dst_hbm, out_hbm, acc_sh, dst_v, src_a, src_b,
               rows_a, rows_b, sem_ia, sem_ib, sem_a, sem_b, ssem_a, ssem_b):
        c = lax.axis_index("c")
        s = lax.axis_index("s")
        w = s * 2 + c

        zvec = jnp.zeros((16,), jnp.float32)

        def zrow(i, _):
            for j in range(8):
                rows_a[i, pl.ds(j * 16, 16)] = zvec
            return 0

        lax.fori_loop(0, BE, zrow, 0)
        for k in range(9):
            pltpu.sync_copy(rows_a, acc_sh.at[pl.ds(s * RPT + k * 64, 64)])
        pltpu.sync_copy(rows_a.at[pl.ds(0, RPT - 576)],
                        acc_sh.at[pl.ds(s * RPT + 576, RPT - 576)])
        plsc.subcore_barrier()

        pltpu.sync_copy(dst_hbm.at[pl.ds(w * NB, NB)], dst_v)

        base = w * EPT_PAD
        pltpu.async_copy(src_hbm.at[pl.ds(base, BE)], src_a, sem_ia)

        def body(b, _):
            even = (b % 2) == 0
            odd = jnp.logical_not(even)
            bm1 = jnp.maximum(b - 1, 0)
            bp1 = jnp.minimum(b + 1, NB - 1)

            @pl.when(jnp.logical_and(even, b < NB))
            def _():
                pltpu.make_async_copy(src_hbm.at[pl.ds(base, BE)], src_a,
                                      sem_ia).wait()

            @pl.when(jnp.logical_and(even, jnp.logical_and(b >= 2, b < NB)))
            def _():
                pltpu.make_async_copy(rows_a, acc_sh.at[dst_v.at[bm1]],
                                      ssem_a).wait()

            @pl.when(jnp.logical_and(even, b < NB))
            def _():
                pltpu.async_copy(h_hbm.at[src_a], rows_a, sem_a)

            @pl.when(jnp.logical_and(even, b >= 1))
            def _():
                pltpu.make_async_copy(h_hbm.at[src_b], rows_b, sem_b).wait()

            @pl.when(jnp.logical_and(even, b < NB))
            def _():
                pltpu.async_copy(src_hbm.at[pl.ds(base + bp1 * BE, BE)],
                                 src_b, sem_ib)

            @pl.when(jnp.logical_and(even, b >= 1))
            def _():
                pltpu.async_copy(rows_b, acc_sh.at[dst_v.at[bm1]], ssem_b,
                                 add=True)

            @pl.when(odd)
            def _():
                pltpu.make_async_copy(src_hbm.at[pl.ds(base, BE)], src_b,
                                      sem_ib).wait()

            @pl.when(jnp.logical_and(odd, b >= 3))
            def _():
                pltpu.make_async_copy(rows_b, acc_sh.at[dst_v.at[bm1]],
                                      ssem_b).wait()

            @pl.when(odd)
            def _():
                pltpu.async_copy(h_hbm.at[src_b], rows_b, sem_b)
                pltpu.make_async_copy(h_hbm.at[src_a], rows_a, sem_a).wait()
                pltpu.async_copy(src_hbm.at[pl.ds(base + bp1 * BE, BE)],
                                 src_a, sem_ia)
                pltpu.async_copy(rows_a, acc_sh.at[dst_v.at[bm1]], ssem_a,
                                 add=True)

            return 0

        lax.fori_loop(0, NB + 1, body, 0)
        pltpu.make_async_copy(src_hbm.at[pl.ds(base, BE)], src_a, sem_ia).wait()
        pltpu.make_async_copy(rows_a, acc_sh.at[dst_v.at[0]], ssem_a).wait()
        pltpu.make_async_copy(rows_b, acc_sh.at[dst_v.at[0]], ssem_b).wait()
        plsc.subcore_barrier()

        pltpu.sync_copy(acc_sh.at[pl.ds(s * RPT, RPT)],
                        out_hbm.at[c, pl.ds(s * RPT, RPT)])

    return segsum(h, srcp, dstp)


def _tc_layer_body(p0, p1, h, wr, wt, b, o, *, relu):
    agg = p0[0] + p1[0]
    acc = jnp.dot(agg, wr[...], preferred_element_type=jnp.float32)
    acc = acc + jnp.dot(h[...], wt[...], preferred_element_type=jnp.float32)
    acc = acc + b[...]
    if relu:
        acc = jnp.maximum(acc, 0.0)
    o[...] = acc


def _tc_layer(p, h, w_rel_t, w_root_t, b, relu):
    row = lambda i: (i, 0)
    rep = lambda i: (0, 0)
    return pl.pallas_call(
        functools.partial(_tc_layer_body, relu=relu),
        grid=(GRID,),
        in_specs=[
            pl.BlockSpec((1, RB, 128), lambda i: (0, i, 0)),
            pl.BlockSpec((1, RB, 128), lambda i: (1, i, 0)),
            pl.BlockSpec((RB, 128), row),
            pl.BlockSpec((128, 128), rep),
            pl.BlockSpec((128, 128), rep),
            pl.BlockSpec((1, 128), rep),
        ],
        out_specs=pl.BlockSpec((RB, 128), row),
        out_shape=jax.ShapeDtypeStruct((N, 128), jnp.float32),
    )(p, p, h, w_rel_t, w_root_t, b)


def _tc_final_body(p0, p1, h, batch_r, wr, wt, b, wl, bl, o, pooled, cnt):
    i = pl.program_id(0)
    h3 = (jnp.dot(p0[0] + p1[0], wr[...], preferred_element_type=jnp.float32)
          + jnp.dot(h[...], wt[...], preferred_element_type=jnp.float32)
          + b[...])
    bm = batch_r[0, 0, :]
    ids = lax.broadcasted_iota(jnp.int32, (G, RB), 0)
    onehot = (bm[None, :] == ids).astype(jnp.float32)

    @pl.when(i == 0)
    def _():
        pooled[...] = jnp.zeros_like(pooled)
        cnt[...] = jnp.zeros_like(cnt)

    pooled[...] += jnp.dot(onehot, h3, preferred_element_type=jnp.float32)
    cnt[...] += jnp.sum(onehot, axis=1, keepdims=True)

    @pl.when(i == pl.num_programs(0) - 1)
    def _():
        mean = pooled[...] / jnp.maximum(cnt[...], 1.0)
        o[...] = jnp.dot(mean, wl[...], preferred_element_type=jnp.float32) + bl[...]


def _tc_final(p, h, batch3, w_rel_t, w_root_t, b, wl, bl):
    row = lambda i: (i, 0)
    rep = lambda i: (0, 0)
    return pl.pallas_call(
        _tc_final_body,
        grid=(GRID,),
        in_specs=[
            pl.BlockSpec((1, RB, 128), lambda i: (0, i, 0)),
            pl.BlockSpec((1, RB, 128), lambda i: (1, i, 0)),
            pl.BlockSpec((RB, 128), row),
            pl.BlockSpec((1, 1, RB), lambda i: (i, 0, 0)),
            pl.BlockSpec((128, 128), rep),
            pl.BlockSpec((128, 128), rep),
            pl.BlockSpec((1, 128), rep),
            pl.BlockSpec((128, 128), rep),
            pl.BlockSpec((1, 128), rep),
        ],
        out_specs=pl.BlockSpec((G, 128), rep),
        out_shape=jax.ShapeDtypeStruct((G, 128), jnp.float32),
        scratch_shapes=[
            pltpu.VMEM((G, 128), jnp.float32),
            pltpu.VMEM((G, 128), jnp.float32),
        ],
    )(p, p, h, batch3, w_rel_t, w_root_t, b, wl, bl)


def kernel(x, edge_index, batch, W_rel1, b_rel1, W_root1, W_rel2, b_rel2,
           W_root2, W_rel3, b_rel3, W_root3, W_lin, b_lin):
    src = edge_index[0].astype(jnp.int32)
    dst = edge_index[1].astype(jnp.int32)

    pad = EPT_PAD - EPT
    srcp = jnp.concatenate(
        [src.reshape(NUM_WORKERS, EPT),
         jnp.zeros((NUM_WORKERS, pad), jnp.int32)], axis=1
    ).reshape(-1)
    dstp = jnp.concatenate(
        [dst.reshape(NUM_WORKERS, EPT),
         jnp.full((NUM_WORKERS, pad), N, jnp.int32)], axis=1
    ).reshape(NUM_WORKERS * NB, BE)

    batch3 = batch.astype(jnp.int32).reshape(GRID, 1, RB)

    wl = jnp.zeros((128, 128), jnp.float32).at[:, :W_lin.shape[0]].set(W_lin.T)
    bl = jnp.zeros((1, 128), jnp.float32).at[0, :b_lin.shape[0]].set(b_lin)

    h = x
    p = _sc_segment_sum(h, srcp, dstp)
    h = _tc_layer(p, h, W_rel1.T, W_root1.T, b_rel1.reshape(1, 128), relu=True)
    p = _sc_segment_sum(h, srcp, dstp)
    h = _tc_layer(p, h, W_rel2.T, W_root2.T, b_rel2.reshape(1, 128), relu=True)
    p = _sc_segment_sum(h, srcp, dstp)
    out = _tc_final(p, h, batch3, W_rel3.T, W_root3.T,
                    b_rel3.reshape(1, 128), wl, bl)
    return out[:, :W_lin.shape[0]]


# speedup vs baseline: 1.0629x; 1.0629x over previous
import functools

import jax
import jax.numpy as jnp
from jax import lax
from jax.experimental import pallas as pl
from jax.experimental.pallas import tpu as pltpu
from jax.experimental.pallas import tpu_sc as plsc

N = 10000
E = 320000
D = 128
G = 64

NUM_WORKERS = 32
BE = 128
EPT = E // NUM_WORKERS
NB = 80
EPT_PAD = NB * BE
NPAD = 10112
RPT = NPAD // 16

RB = 1000
GRID = N // RB


def _sc_segment_sum(h, srcp, dstp):
    mesh = plsc.VectorSubcoreMesh(core_axis_name="c", subcore_axis_name="s")

    @functools.partial(
        pl.kernel,
        out_type=jax.ShapeDtypeStruct((2, NPAD, 128), jnp.float32),
        mesh=mesh,
        scratch_types=[
            pltpu.VMEM_SHARED((NPAD, 128), jnp.float32),
            pltpu.VMEM((NB, BE), jnp.int32),
            pltpu.VMEM((BE,), jnp.int32),
            pltpu.VMEM((BE,), jnp.int32),
            pltpu.VMEM((BE, 128), jnp.float32),
            pltpu.VMEM((BE, 128), jnp.float32),
            pltpu.SemaphoreType.DMA,
            pltpu.SemaphoreType.DMA,
            pltpu.SemaphoreType.DMA,
            pltpu.SemaphoreType.DMA,
            pltpu.SemaphoreType.DMA,
            pltpu.SemaphoreType.DMA,
        ],
    )
    def segsum(h_hbm, src_hbm, dst_hbm, out_hbm, acc_sh, dst_v, src_a, src_b,
               rows_a, rows_b, sem_ia, sem_ib, sem_a, sem_b, ssem_a, ssem_b):
        c = lax.axis_index("c")
        s = lax.axis_index("s")
        w = s * 2 + c

        zvec = jnp.zeros((16,), jnp.float32)

        def zrow(i, _):
            for j in range(8):
                rows_a[i, pl.ds(j * 16, 16)] = zvec
            return 0

        lax.fori_loop(0, BE, zrow, 0)
        for k in range(4):
            pltpu.sync_copy(rows_a, acc_sh.at[pl.ds(s * RPT + k * 128, 128)])
        pltpu.sync_copy(rows_a.at[pl.ds(0, RPT - 512)],
                        acc_sh.at[pl.ds(s * RPT + 512, RPT - 512)])
        plsc.subcore_barrier()

        pltpu.sync_copy(dst_hbm.at[pl.ds(w * NB, NB)], dst_v)

        base = w * EPT_PAD
        pltpu.async_copy(src_hbm.at[pl.ds(base, BE)], src_a, sem_ia)

        pltpu.make_async_copy(src_hbm.at[pl.ds(base, BE)], src_a,
                              sem_ia).wait()
        pltpu.async_copy(h_hbm.at[src_a], rows_a, sem_a)
        pltpu.async_copy(src_hbm.at[pl.ds(base + BE, BE)], src_b, sem_ib)

        pltpu.make_async_copy(src_hbm.at[pl.ds(base, BE)], src_b,
                              sem_ib).wait()
        pltpu.async_copy(h_hbm.at[src_b], rows_b, sem_b)
        pltpu.make_async_copy(h_hbm.at[src_a], rows_a, sem_a).wait()
        pltpu.async_copy(src_hbm.at[pl.ds(base + 2 * BE, BE)], src_a, sem_ia)
        pltpu.async_copy(rows_a, acc_sh.at[dst_v.at[0]], ssem_a, add=True)

        def pair(g, _):
            be = 2 * g
            bo = be + 1
            bo_next = jnp.minimum(bo + 1, NB - 1)

            pltpu.make_async_copy(src_hbm.at[pl.ds(base, BE)], src_a,
                                  sem_ia).wait()
            pltpu.make_async_copy(rows_a, acc_sh.at[dst_v.at[be]],
                                  ssem_a).wait()
            pltpu.async_copy(h_hbm.at[src_a], rows_a, sem_a)
            pltpu.make_async_copy(h_hbm.at[src_b], rows_b, sem_b).wait()
            pltpu.async_copy(src_hbm.at[pl.ds(base + bo * BE, BE)], src_b,
                             sem_ib)
            pltpu.async_copy(rows_b, acc_sh.at[dst_v.at[be - 1]], ssem_b,
                             add=True)

            pltpu.make_async_copy(src_hbm.at[pl.ds(base, BE)], src_b,
                                  sem_ib).wait()
            pltpu.make_async_copy(rows_b, acc_sh.at[dst_v.at[bo]],
                                  ssem_b).wait()
            pltpu.async_copy(h_hbm.at[src_b], rows_b, sem_b)
            pltpu.make_async_copy(h_hbm.at[src_a], rows_a, sem_a).wait()
            pltpu.async_copy(src_hbm.at[pl.ds(base + bo_next * BE, BE)],
                             src_a, sem_ia)
            pltpu.async_copy(rows_a, acc_sh.at[dst_v.at[be]], ssem_a,
                             add=True)
            return 0

        lax.fori_loop(1, NB // 2, pair, 0)
        pltpu.make_async_copy(h_hbm.at[src_b], rows_b, sem_b).wait()
        pltpu.async_copy(rows_b, acc_sh.at[dst_v.at[NB - 1]], ssem_b,
                         add=True)
        pltpu.make_async_copy(src_hbm.at[pl.ds(base, BE)], src_a, sem_ia).wait()
        pltpu.make_async_copy(rows_a, acc_sh.at[dst_v.at[0]], ssem_a).wait()
        pltpu.make_async_copy(rows_b, acc_sh.at[dst_v.at[0]], ssem_b).wait()
        plsc.subcore_barrier()

        pltpu.sync_copy(acc_sh.at[pl.ds(s * RPT, RPT)],
                        out_hbm.at[c, pl.ds(s * RPT, RPT)])

    return segsum(h, srcp, dstp)


def _tc_layer_body(p0, p1, h, wr, wt, b, o, *, relu):
    agg = p0[0] + p1[0]
    acc = jnp.dot(agg, wr[...], preferred_element_type=jnp.float32)
    acc = acc + jnp.dot(h[...], wt[...], preferred_element_type=jnp.float32)
    acc = acc + b[...]
    if relu:
        acc = jnp.maximum(acc, 0.0)
    o[...] = acc


def _tc_layer(p, h, w_rel_t, w_root_t, b, relu):
    row = lambda i: (i, 0)
    rep = lambda i: (0, 0)
    return pl.pallas_call(
        functools.partial(_tc_layer_body, relu=relu),
        grid=(GRID,),
        in_specs=[
            pl.BlockSpec((1, RB, 128), lambda i: (0, i, 0)),
            pl.BlockSpec((1, RB, 128), lambda i: (1, i, 0)),
            pl.BlockSpec((RB, 128), row),
            pl.BlockSpec((128, 128), rep),
            pl.BlockSpec((128, 128), rep),
            pl.BlockSpec((1, 128), rep),
        ],
        out_specs=pl.BlockSpec((RB, 128), row),
        out_shape=jax.ShapeDtypeStruct((N, 128), jnp.float32),
    )(p, p, h, w_rel_t, w_root_t, b)


def _tc_final_body(p0, p1, h, batch_r, wr, wt, b, wl, bl, o, pooled, cnt):
    i = pl.program_id(0)
    h3 = (jnp.dot(p0[0] + p1[0], wr[...], preferred_element_type=jnp.float32)
          + jnp.dot(h[...], wt[...], preferred_element_type=jnp.float32)
          + b[...])
    bm = batch_r[0, 0, :]
    ids = lax.broadcasted_iota(jnp.int32, (G, RB), 0)
    onehot = (bm[None, :] == ids).astype(jnp.float32)

    @pl.when(i == 0)
    def _():
        pooled[...] = jnp.zeros_like(pooled)
        cnt[...] = jnp.zeros_like(cnt)

    pooled[...] += jnp.dot(onehot, h3, preferred_element_type=jnp.float32)
    cnt[...] += jnp.sum(onehot, axis=1, keepdims=True)

    @pl.when(i == pl.num_programs(0) - 1)
    def _():
        mean = pooled[...] / jnp.maximum(cnt[...], 1.0)
        o[...] = jnp.dot(mean, wl[...], preferred_element_type=jnp.float32) + bl[...]


def _tc_final(p, h, batch3, w_rel_t, w_root_t, b, wl, bl):
    row = lambda i: (i, 0)
    rep = lambda i: (0, 0)
    return pl.pallas_call(
        _tc_final_body,
        grid=(GRID,),
        in_specs=[
            pl.BlockSpec((1, RB, 128), lambda i: (0, i, 0)),
            pl.BlockSpec((1, RB, 128), lambda i: (1, i, 0)),
            pl.BlockSpec((RB, 128), row),
            pl.BlockSpec((1, 1, RB), lambda i: (i, 0, 0)),
            pl.BlockSpec((128, 128), rep),
            pl.BlockSpec((128, 128), rep),
            pl.BlockSpec((1, 128), rep),
            pl.BlockSpec((128, 128), rep),
            pl.BlockSpec((1, 128), rep),
        ],
        out_specs=pl.BlockSpec((G, 128), rep),
        out_shape=jax.ShapeDtypeStruct((G, 128), jnp.float32),
        scratch_shapes=[
            pltpu.VMEM((G, 128), jnp.float32),
            pltpu.VMEM((G, 128), jnp.float32),
        ],
    )(p, p, h, batch3, w_rel_t, w_root_t, b, wl, bl)


def kernel(x, edge_index, batch, W_rel1, b_rel1, W_root1, W_rel2, b_rel2,
           W_root2, W_rel3, b_rel3, W_root3, W_lin, b_lin):
    src = edge_index[0].astype(jnp.int32)
    dst = edge_index[1].astype(jnp.int32)

    pad = EPT_PAD - EPT
    srcp = jnp.concatenate(
        [src.reshape(NUM_WORKERS, EPT),
         jnp.zeros((NUM_WORKERS, pad), jnp.int32)], axis=1
    ).reshape(-1)
    dstp = jnp.concatenate(
        [dst.reshape(NUM_WORKERS, EPT),
         jnp.full((NUM_WORKERS, pad), N, jnp.int32)], axis=1
    ).reshape(NUM_WORKERS * NB, BE)

    batch3 = batch.astype(jnp.int32).reshape(GRID, 1, RB)

    wl = jnp.zeros((128, 128), jnp.float32).at[:, :W_lin.shape[0]].set(W_lin.T)
    bl = jnp.zeros((1, 128), jnp.float32).at[0, :b_lin.shape[0]].set(b_lin)

    h = x
    p = _sc_segment_sum(h, srcp, dstp)
    h = _tc_layer(p, h, W_rel1.T, W_root1.T, b_rel1.reshape(1, 128), relu=True)
    p = _sc_segment_sum(h, srcp, dstp)
    h = _tc_layer(p, h, W_rel2.T, W_root2.T, b_rel2.reshape(1, 128), relu=True)
    p = _sc_segment_sum(h, srcp, dstp)
    out = _tc_final(p, h, batch3, W_rel3.T, W_root3.T,
                    b_rel3.reshape(1, 128), wl, bl)
    return out[:, :W_lin.shape[0]]


# speedup vs baseline: 1.0667x; 1.0035x over previous
import functools

import jax
import jax.numpy as jnp
from jax import lax
from jax.experimental import pallas as pl
from jax.experimental.pallas import tpu as pltpu
from jax.experimental.pallas import tpu_sc as plsc

N = 10000
E = 320000
D = 128
G = 64

NUM_WORKERS = 32
BE = 128
EPT = E // NUM_WORKERS
NB = 80
EPT_PAD = NB * BE
NPAD = 10112
RPT = NPAD // 16

RB = 1000
GRID = N // RB


def _sc_segment_sum(h, srcp, dstp):
    mesh = plsc.VectorSubcoreMesh(core_axis_name="c", subcore_axis_name="s")

    @functools.partial(
        pl.kernel,
        out_type=jax.ShapeDtypeStruct((2, NPAD, 128), jnp.float32),
        mesh=mesh,
        scratch_types=[
            pltpu.VMEM_SHARED((NPAD, 128), jnp.float32),
            pltpu.VMEM((NB, BE), jnp.int32),
            pltpu.VMEM((BE,), jnp.int32),
            pltpu.VMEM((BE,), jnp.int32),
            pltpu.VMEM((BE, 128), jnp.float32),
            pltpu.VMEM((BE, 128), jnp.float32),
            pltpu.SemaphoreType.DMA,
            pltpu.SemaphoreType.DMA,
            pltpu.SemaphoreType.DMA,
            pltpu.SemaphoreType.DMA,
            pltpu.SemaphoreType.DMA,
            pltpu.SemaphoreType.DMA,
        ],
    )
    def segsum(h_hbm, src_hbm, dst_hbm, out_hbm, acc_sh, dst_v, src_a, src_b,
               rows_a, rows_b, sem_ia, sem_ib, sem_a, sem_b, ssem_a, ssem_b):
        c = lax.axis_index("c")
        s = lax.axis_index("s")
        w = s * 2 + c

        base = w * EPT_PAD
        pltpu.async_copy(src_hbm.at[pl.ds(base, BE)], src_a, sem_ia)
        pltpu.sync_copy(dst_hbm.at[pl.ds(w * NB, NB)], dst_v)
        pltpu.make_async_copy(src_hbm.at[pl.ds(base, BE)], src_a,
                              sem_ia).wait()
        pltpu.async_copy(h_hbm.at[src_a], rows_a, sem_a)
        pltpu.async_copy(src_hbm.at[pl.ds(base + BE, BE)], src_b, sem_ib)

        zvec = jnp.zeros((16,), jnp.float32)

        def zrow(i, _):
            for j in range(8):
                rows_b[i, pl.ds(j * 16, 16)] = zvec
            return 0

        lax.fori_loop(0, BE, zrow, 0)
        for k in range(4):
            pltpu.sync_copy(rows_b, acc_sh.at[pl.ds(s * RPT + k * 128, 128)])
        pltpu.sync_copy(rows_b.at[pl.ds(0, RPT - 512)],
                        acc_sh.at[pl.ds(s * RPT + 512, RPT - 512)])
        plsc.subcore_barrier()

        pltpu.make_async_copy(src_hbm.at[pl.ds(base, BE)], src_b,
                              sem_ib).wait()
        pltpu.async_copy(h_hbm.at[src_b], rows_b, sem_b)
        pltpu.make_async_copy(h_hbm.at[src_a], rows_a, sem_a).wait()
        pltpu.async_copy(src_hbm.at[pl.ds(base + 2 * BE, BE)], src_a, sem_ia)
        pltpu.async_copy(rows_a, acc_sh.at[dst_v.at[0]], ssem_a, add=True)

        def pair(g, _):
            be = 2 * g
            bo = be + 1
            bo_next = jnp.minimum(bo + 1, NB - 1)

            pltpu.make_async_copy(src_hbm.at[pl.ds(base, BE)], src_a,
                                  sem_ia).wait()
            pltpu.make_async_copy(rows_a, acc_sh.at[dst_v.at[be]],
                                  ssem_a).wait()
            pltpu.async_copy(h_hbm.at[src_a], rows_a, sem_a)
            pltpu.make_async_copy(h_hbm.at[src_b], rows_b, sem_b).wait()
            pltpu.async_copy(src_hbm.at[pl.ds(base + bo * BE, BE)], src_b,
                             sem_ib)
            pltpu.async_copy(rows_b, acc_sh.at[dst_v.at[be - 1]], ssem_b,
                             add=True)

            pltpu.make_async_copy(src_hbm.at[pl.ds(base, BE)], src_b,
                                  sem_ib).wait()
            pltpu.make_async_copy(rows_b, acc_sh.at[dst_v.at[bo]],
                                  ssem_b).wait()
            pltpu.async_copy(h_hbm.at[src_b], rows_b, sem_b)
            pltpu.make_async_copy(h_hbm.at[src_a], rows_a, sem_a).wait()
            pltpu.async_copy(src_hbm.at[pl.ds(base + bo_next * BE, BE)],
                             src_a, sem_ia)
            pltpu.async_copy(rows_a, acc_sh.at[dst_v.at[be]], ssem_a,
                             add=True)
            return 0

        lax.fori_loop(1, NB // 2, pair, 0)
        pltpu.make_async_copy(h_hbm.at[src_b], rows_b, sem_b).wait()
        pltpu.async_copy(rows_b, acc_sh.at[dst_v.at[NB - 1]], ssem_b,
                         add=True)
        pltpu.make_async_copy(src_hbm.at[pl.ds(base, BE)], src_a, sem_ia).wait()
        pltpu.make_async_copy(rows_a, acc_sh.at[dst_v.at[0]], ssem_a).wait()
        pltpu.make_async_copy(rows_b, acc_sh.at[dst_v.at[0]], ssem_b).wait()
        plsc.subcore_barrier()

        pltpu.sync_copy(acc_sh.at[pl.ds(s * RPT, RPT)],
                        out_hbm.at[c, pl.ds(s * RPT, RPT)])

    return segsum(h, srcp, dstp)


def _tc_layer_body(p0, p1, h, wr, wt, b, o, *, relu):
    agg = p0[0] + p1[0]
    acc = jnp.dot(agg, wr[...], preferred_element_type=jnp.float32)
    acc = acc + jnp.dot(h[...], wt[...], preferred_element_type=jnp.float32)
    acc = acc + b[...]
    if relu:
        acc = jnp.maximum(acc, 0.0)
    o[...] = acc


def _tc_layer(p, h, w_rel_t, w_root_t, b, relu):
    row = lambda i: (i, 0)
    rep = lambda i: (0, 0)
    return pl.pallas_call(
        functools.partial(_tc_layer_body, relu=relu),
        grid=(GRID,),
        in_specs=[
            pl.BlockSpec((1, RB, 128), lambda i: (0, i, 0)),
            pl.BlockSpec((1, RB, 128), lambda i: (1, i, 0)),
            pl.BlockSpec((RB, 128), row),
            pl.BlockSpec((128, 128), rep),
            pl.BlockSpec((128, 128), rep),
            pl.BlockSpec((1, 128), rep),
        ],
        out_specs=pl.BlockSpec((RB, 128), row),
        out_shape=jax.ShapeDtypeStruct((N, 128), jnp.float32),
    )(p, p, h, w_rel_t, w_root_t, b)


def _tc_final_body(p0, p1, h, batch_r, wr, wt, b, wl, bl, o, pooled, cnt):
    i = pl.program_id(0)
    h3 = (jnp.dot(p0[0] + p1[0], wr[...], preferred_element_type=jnp.float32)
          + jnp.dot(h[...], wt[...], preferred_element_type=jnp.float32)
          + b[...])
    bm = batch_r[0, 0, :]
    ids = lax.broadcasted_iota(jnp.int32, (G, RB), 0)
    onehot = (bm[None, :] == ids).astype(jnp.float32)

    @pl.when(i == 0)
    def _():
        pooled[...] = jnp.zeros_like(pooled)
        cnt[...] = jnp.zeros_like(cnt)

    pooled[...] += jnp.dot(onehot, h3, preferred_element_type=jnp.float32)
    cnt[...] += jnp.sum(onehot, axis=1, keepdims=True)

    @pl.when(i == pl.num_programs(0) - 1)
    def _():
        mean = pooled[...] / jnp.maximum(cnt[...], 1.0)
        o[...] = jnp.dot(mean, wl[...], preferred_element_type=jnp.float32) + bl[...]


def _tc_final(p, h, batch3, w_rel_t, w_root_t, b, wl, bl):
    row = lambda i: (i, 0)
    rep = lambda i: (0, 0)
    return pl.pallas_call(
        _tc_final_body,
        grid=(GRID,),
        in_specs=[
            pl.BlockSpec((1, RB, 128), lambda i: (0, i, 0)),
            pl.BlockSpec((1, RB, 128), lambda i: (1, i, 0)),
            pl.BlockSpec((RB, 128), row),
            pl.BlockSpec((1, 1, RB), lambda i: (i, 0, 0)),
            pl.BlockSpec((128, 128), rep),
            pl.BlockSpec((128, 128), rep),
            pl.BlockSpec((1, 128), rep),
            pl.BlockSpec((128, 128), rep),
            pl.BlockSpec((1, 128), rep),
        ],
        out_specs=pl.BlockSpec((G, 128), rep),
        out_shape=jax.ShapeDtypeStruct((G, 128), jnp.float32),
        scratch_shapes=[
            pltpu.VMEM((G, 128), jnp.float32),
            pltpu.VMEM((G, 128), jnp.float32),
        ],
    )(p, p, h, batch3, w_rel_t, w_root_t, b, wl, bl)


def kernel(x, edge_index, batch, W_rel1, b_rel1, W_root1, W_rel2, b_rel2,
           W_root2, W_rel3, b_rel3, W_root3, W_lin, b_lin):
    src = edge_index[0].astype(jnp.int32)
    dst = edge_index[1].astype(jnp.int32)

    pad = EPT_PAD - EPT
    srcp = jnp.concatenate(
        [src.reshape(NUM_WORKERS, EPT),
         jnp.zeros((NUM_WORKERS, pad), jnp.int32)], axis=1
    ).reshape(-1)
    dstp = jnp.concatenate(
        [dst.reshape(NUM_WORKERS, EPT),
         jnp.full((NUM_WORKERS, pad), N, jnp.int32)], axis=1
    ).reshape(NUM_WORKERS * NB, BE)

    batch3 = batch.astype(jnp.int32).reshape(GRID, 1, RB)

    wl = jnp.zeros((128, 128), jnp.float32).at[:, :W_lin.shape[0]].set(W_lin.T)
    bl = jnp.zeros((1, 128), jnp.float32).at[0, :b_lin.shape[0]].set(b_lin)

    h = x
    p = _sc_segment_sum(h, srcp, dstp)
    h = _tc_layer(p, h, W_rel1.T, W_root1.T, b_rel1.reshape(1, 128), relu=True)
    p = _sc_segment_sum(h, srcp, dstp)
    h = _tc_layer(p, h, W_rel2.T, W_root2.T, b_rel2.reshape(1, 128), relu=True)
    p = _sc_segment_sum(h, srcp, dstp)
    out = _tc_final(p, h, batch3, W_rel3.T, W_root3.T,
                    b_rel3.reshape(1, 128), wl, bl)
    return out[:, :W_lin.shape[0]]
